# Initial kernel scaffold; baseline (speedup 1.0000x reference)
import jax
import jax.numpy as jnp
from jax.experimental import pallas as pl


def kernel(x, edge_index, W, b, att):
    raise NotImplementedError("write your pallas kernel here")


# speedup vs baseline: 9.7856x; 9.7856x over previous
import functools

import jax
import jax.numpy as jnp
from jax import lax
from jax.experimental import pallas as pl
from jax.experimental.pallas import tpu as pltpu
from jax.experimental.pallas import tpu_sc as plsc

N_SC = 2
N_SUB = 16
LANES = 16


def _lin_body(x_ref, w_ref, b_ref, att_ref, h_ref, a_ref):
    x = x_ref[...]
    h = lax.dot_general(x, w_ref[...], (((1,), (1,)), ((), ())),
                        preferred_element_type=jnp.float32)
    h = h + b_ref[...]
    h_ref[...] = h
    a_ref[...] = jnp.sum(h * att_ref[...], axis=1, keepdims=True)


def _linear_and_scores(x_p, W, b, att, n_pad, d):
    rb = 1280
    grid = (n_pad // rb,)
    return pl.pallas_call(
        _lin_body,
        grid=grid,
        in_specs=[
            pl.BlockSpec((rb, d), lambda i: (i, 0)),
            pl.BlockSpec((d, d), lambda i: (0, 0)),
            pl.BlockSpec((1, d), lambda i: (0, 0)),
            pl.BlockSpec((1, d), lambda i: (0, 0)),
        ],
        out_specs=[
            pl.BlockSpec((rb, d), lambda i: (i, 0)),
            pl.BlockSpec((rb, 1), lambda i: (i, 0)),
        ],
        out_shape=[
            jax.ShapeDtypeStruct((n_pad, d), jnp.float32),
            jax.ShapeDtypeStruct((n_pad, 1), jnp.float32),
        ],
    )(x_p, W, b.reshape(1, d), att.reshape(1, d))


def _fin_body(p_ref, o_ref):
    o_ref[...] = jnp.maximum(p_ref[0] + p_ref[1], 0.0)


def _combine(partials, n_acc, d):
    rb = n_acc // 8
    return pl.pallas_call(
        _fin_body,
        grid=(n_acc // rb,),
        in_specs=[pl.BlockSpec((2, rb, d), lambda i: (0, i, 0))],
        out_specs=pl.BlockSpec((rb, d), lambda i: (i, 0)),
        out_shape=jax.ShapeDtypeStruct((n_acc, d), jnp.float32),
    )(partials)


SA = 3072
SB = 512


def _make_sc_kernel(n_pad, n_acc, d, e_pad):
    ch_a = e_pad // N_SUB
    ch_b = e_pad // (N_SUB * N_SC)
    den_rows = n_pad // 128
    zrows = n_acc // N_SUB
    mesh = plsc.VectorSubcoreMesh(core_axis_name="c", subcore_axis_name="s")

    @functools.partial(
        pl.kernel,
        out_type=jax.ShapeDtypeStruct((N_SC, n_acc, d), jnp.float32),
        mesh=mesh,
        compiler_params=pltpu.CompilerParams(needs_layout_passes=False),
        scratch_types=[
            pltpu.VMEM((n_pad,), jnp.float32),
            pltpu.VMEM((den_rows, 128), jnp.float32),
            pltpu.VMEM((SA,), jnp.int32),
            pltpu.VMEM((SA,), jnp.int32),
            pltpu.VMEM((SB,), jnp.float32),
            pltpu.VMEM((128, d), jnp.float32),
            pltpu.VMEM_SHARED((n_acc, d), jnp.float32),
            pltpu.VMEM_SHARED((den_rows, 128), jnp.float32),
            pltpu.SemaphoreType.DMA,
        ],
    )
    def sc_kernel(h_hbm, a_hbm, src_hbm, dst_hbm, out_hbm,
                  a_loc, den_loc, src_loc, dst_loc, w_loc, rows,
                  out_acc, den_sh, sem):
        c = lax.axis_index("c")
        s = lax.axis_index("s")
        z16 = jnp.zeros((LANES,), jnp.float32)

        pltpu.sync_copy(a_hbm, a_loc)

        def zden(i, carry):
            for j in range(128 // LANES):
                den_loc[i, pl.ds(j * LANES, LANES)] = z16
            return carry
        lax.fori_loop(0, den_rows, zden, 0)

        def zrow(i, carry):
            for j in range(d // LANES):
                rows[i, pl.ds(j * LANES, LANES)] = z16
            return carry
        lax.fori_loop(0, 128, zrow, 0)

        @pl.when(s == 0)
        def _():
            pltpu.sync_copy(den_loc, den_sh)

        for k in range(zrows // 128):
            pltpu.sync_copy(rows, out_acc.at[pl.ds(s * zrows + k * 128, 128)])
        rem = zrows % 128
        if rem:
            pltpu.sync_copy(rows.at[pl.ds(0, rem)],
                            out_acc.at[pl.ds(s * zrows + zrows - rem, rem)])

        def edge_vals(i):
            si = src_loc[pl.ds(i * LANES, LANES)]
            di = dst_loc[pl.ds(i * LANES, LANES)]
            av = plsc.load_gather(a_loc, [si]) + plsc.load_gather(a_loc, [di])
            al = jnp.where(av >= 0.0, av, 0.2 * av)
            e = jnp.exp(al)
            r = lax.shift_right_logical(si, 7)
            lane = jnp.bitwise_and(si, 127)
            return e, r, lane

        def pa_blk(t, carry):
            base = s * ch_a + t * SA
            pltpu.sync_copy(src_hbm.at[pl.ds(base, SA)], src_loc)
            pltpu.sync_copy(dst_hbm.at[pl.ds(base, SA)], dst_loc)

            def pa(i, carry2):
                e, r, lane = edge_vals(i)
                plsc.addupdate_scatter(den_loc, [r, lane], e)
                return carry2
            lax.fori_loop(0, SA // LANES, pa, 0)
            return carry
        lax.fori_loop(0, ch_a // SA, pa_blk, 0)

        plsc.subcore_barrier()
        for k in range(den_rows // LANES):
            ivec = lax.iota(jnp.int32, LANES) + k * LANES
            pltpu.sync_copy(den_loc.at[pl.ds(k * LANES, LANES)],
                            den_sh.at[ivec], add=True)
        plsc.subcore_barrier()
        pltpu.sync_copy(den_sh, den_loc)

        wid = s * N_SC + c

        def pb_blk(t, carry):
            base = wid * ch_b + t * SB
            pltpu.sync_copy(src_hbm.at[pl.ds(base, SB)], src_loc.at[pl.ds(0, SB)])
            pltpu.sync_copy(dst_hbm.at[pl.ds(base, SB)], dst_loc.at[pl.ds(0, SB)])

            def wv_loop(i, carry2):
                e, r, lane = edge_vals(i)
                den = plsc.load_gather(den_loc, [r, lane])
                w_loc[pl.ds(i * LANES, LANES)] = e / (den + 1e-16)
                return carry2
            lax.fori_loop(0, SB // LANES, wv_loop, 0)

            def sub(u, carry2):
                pltpu.sync_copy(h_hbm.at[src_loc.at[pl.ds(u * 128, 128)]], rows)

                def scale(g, carry3):
                    wv16 = w_loc[pl.ds(u * 128 + g * LANES, LANES)]
                    for k in range(LANES):
                        wv = wv16[k]
                        for j in range(d // LANES):
                            sl = pl.ds(j * LANES, LANES)
                            rows[g * LANES + k, sl] = rows[g * LANES + k, sl] * wv
                    return carry3
                lax.fori_loop(0, 128 // LANES, scale, 0)

                cps = []
                for k in range(128 // LANES):
                    dvec = dst_loc[pl.ds(u * 128 + k * LANES, LANES)]
                    cps.append(pltpu.async_copy(
                        rows.at[pl.ds(k * LANES, LANES)],
                        out_acc.at[dvec], sem, add=True))
                for cp in cps:
                    cp.wait()
                return carry2
            lax.fori_loop(0, SB // 128, sub, 0)
            return carry
        lax.fori_loop(0, ch_b // SB, pb_blk, 0)

        plsc.subcore_barrier()
        pltpu.sync_copy(out_acc.at[pl.ds(s * zrows, zrows)],
                        out_hbm.at[c, pl.ds(s * zrows, zrows)])

    return sc_kernel


def kernel(x, edge_index, W, b, att):
    n, d = x.shape
    e0 = edge_index.shape[1]
    e1 = e0 + n

    n_pad = 10240
    n_acc = 10112
    step = N_SUB * SA
    e_pad = ((e1 + step - 1) // step) * step

    loops = jnp.arange(n, dtype=edge_index.dtype)
    ei = jnp.concatenate([edge_index, jnp.stack([loops, loops])], axis=1)
    pad = jnp.full((e_pad - e1,), n, dtype=jnp.int32)
    src = jnp.concatenate([ei[0], pad])
    dst = jnp.concatenate([ei[1], pad])

    x_p = jnp.concatenate([x, jnp.zeros((n_pad - n, d), x.dtype)])
    h, a2 = _linear_and_scores(x_p, W, b, att, n_pad, d)
    a = a2.reshape(n_pad)

    partials = _make_sc_kernel(n_pad, n_acc, d, e_pad)(h, a, src, dst)
    out = _combine(partials, n_acc, d)
    return (out[:n], edge_index)


# speedup vs baseline: 11.0329x; 1.1275x over previous
import functools

import jax
import jax.numpy as jnp
from jax import lax
from jax.experimental import pallas as pl
from jax.experimental.pallas import tpu as pltpu
from jax.experimental.pallas import tpu_sc as plsc

N_SC = 2
N_SUB = 16
LANES = 16


def _lin_body(x_ref, w_ref, b_ref, att_ref, h_ref, a_ref):
    x = x_ref[...]
    h = lax.dot_general(x, w_ref[...], (((1,), (1,)), ((), ())),
                        preferred_element_type=jnp.float32)
    h = h + b_ref[...]
    h_ref[...] = h
    a_ref[...] = jnp.sum(h * att_ref[...], axis=1, keepdims=True)


def _linear_and_scores(x_p, W, b, att, n_pad, d):
    rb = 1280
    grid = (n_pad // rb,)
    return pl.pallas_call(
        _lin_body,
        grid=grid,
        in_specs=[
            pl.BlockSpec((rb, d), lambda i: (i, 0)),
            pl.BlockSpec((d, d), lambda i: (0, 0)),
            pl.BlockSpec((1, d), lambda i: (0, 0)),
            pl.BlockSpec((1, d), lambda i: (0, 0)),
        ],
        out_specs=[
            pl.BlockSpec((rb, d), lambda i: (i, 0)),
            pl.BlockSpec((rb, 1), lambda i: (i, 0)),
        ],
        out_shape=[
            jax.ShapeDtypeStruct((n_pad, d), jnp.float32),
            jax.ShapeDtypeStruct((n_pad, 1), jnp.float32),
        ],
    )(x_p, W, b.reshape(1, d), att.reshape(1, d))


def _fin_body(p_ref, o_ref):
    o_ref[...] = jnp.maximum(p_ref[0] + p_ref[1], 0.0)


def _combine(partials, n_acc, d):
    rb = n_acc // 8
    return pl.pallas_call(
        _fin_body,
        grid=(n_acc // rb,),
        in_specs=[pl.BlockSpec((2, rb, d), lambda i: (0, i, 0))],
        out_specs=pl.BlockSpec((rb, d), lambda i: (i, 0)),
        out_shape=jax.ShapeDtypeStruct((n_acc, d), jnp.float32),
    )(partials)


SA = 3072
SB = 192


def _make_sc_kernel(n_pad, n_acc, d, e_pad):
    ch_a = e_pad // N_SUB
    ch_b = e_pad // (N_SUB * N_SC)
    den_rows = n_pad // 128
    zrows = n_acc // N_SUB
    mesh = plsc.VectorSubcoreMesh(core_axis_name="c", subcore_axis_name="s")

    @functools.partial(
        pl.kernel,
        out_type=jax.ShapeDtypeStruct((N_SC, n_acc, d), jnp.float32),
        mesh=mesh,
        compiler_params=pltpu.CompilerParams(needs_layout_passes=False),
        scratch_types=[
            pltpu.VMEM((n_pad,), jnp.float32),
            pltpu.VMEM((den_rows, 128), jnp.float32),
            pltpu.VMEM((SA,), jnp.int32),
            pltpu.VMEM((SA,), jnp.int32),
            pltpu.VMEM((SB,), jnp.float32),
            pltpu.VMEM((3, 32, d), jnp.float32),
            pltpu.VMEM_SHARED((n_acc, d), jnp.float32),
            pltpu.VMEM_SHARED((den_rows, 128), jnp.float32),
            pltpu.SemaphoreType.DMA,
            pltpu.SemaphoreType.DMA,
            pltpu.SemaphoreType.DMA,
            pltpu.SemaphoreType.DMA,
            pltpu.SemaphoreType.DMA,
            pltpu.SemaphoreType.DMA,
            pltpu.SemaphoreType.DMA,
        ],
    )
    def sc_kernel(h_hbm, a_hbm, src_hbm, dst_hbm, out_hbm,
                  a_loc, den_loc, src_loc, dst_loc, w_loc, rows3,
                  out_acc, den_sh, sem_st, sem_g0, sem_g1, sem_g2,
                  sem_s0, sem_s1, sem_s2):
        sem_g = (sem_g0, sem_g1, sem_g2)
        sem_s = (sem_s0, sem_s1, sem_s2)
        c = lax.axis_index("c")
        s = lax.axis_index("s")
        z16 = jnp.zeros((LANES,), jnp.float32)

        pltpu.sync_copy(a_hbm, a_loc)

        def zden(i, carry):
            for j in range(128 // LANES):
                den_loc[i, pl.ds(j * LANES, LANES)] = z16
            return carry
        lax.fori_loop(0, den_rows, zden, 0)

        def zrow(i, carry):
            for bb in range(3):
                for j in range(d // LANES):
                    rows3[bb, i, pl.ds(j * LANES, LANES)] = z16
            return carry
        lax.fori_loop(0, 32, zrow, 0)

        @pl.when(s == 0)
        def _():
            pltpu.sync_copy(den_loc, den_sh)

        for k in range(zrows // 32):
            pltpu.sync_copy(rows3.at[0], out_acc.at[pl.ds(s * zrows + k * 32, 32)])
        rem = zrows % 32
        if rem:
            pltpu.sync_copy(rows3.at[0, pl.ds(0, rem)],
                            out_acc.at[pl.ds(s * zrows + zrows - rem, rem)])

        def edge_vals(i):
            si = src_loc[pl.ds(i * LANES, LANES)]
            di = dst_loc[pl.ds(i * LANES, LANES)]
            av = plsc.load_gather(a_loc, [si]) + plsc.load_gather(a_loc, [di])
            al = jnp.where(av >= 0.0, av, 0.2 * av)
            e = jnp.exp(al)
            r = lax.shift_right_logical(si, 7)
            lane = jnp.bitwise_and(si, 127)
            return e, r, lane

        def pa_blk(t, carry):
            base = s * ch_a + t * SA
            pltpu.sync_copy(src_hbm.at[pl.ds(base, SA)], src_loc)
            pltpu.sync_copy(dst_hbm.at[pl.ds(base, SA)], dst_loc)

            def pa(i, carry2):
                e, r, lane = edge_vals(i)
                plsc.addupdate_scatter(den_loc, [r, lane], e)
                return carry2
            lax.fori_loop(0, SA // LANES, pa, 0)
            return carry
        lax.fori_loop(0, ch_a // SA, pa_blk, 0)

        plsc.subcore_barrier()
        for k in range(den_rows // LANES):
            ivec = lax.iota(jnp.int32, LANES) + k * LANES
            pltpu.sync_copy(den_loc.at[pl.ds(k * LANES, LANES)],
                            den_sh.at[ivec], add=True)
        plsc.subcore_barrier()
        pltpu.sync_copy(den_sh, den_loc)

        wid = s * N_SC + c
        n_it = ch_b // SB
        b_base = wid * ch_b

        def stage(it):
            half = lax.rem(it, 2) * SB
            base = b_base + it * SB
            pltpu.async_copy(src_hbm.at[pl.ds(base, SB)],
                             src_loc.at[pl.ds(half, SB)], sem_st)
            pltpu.async_copy(dst_hbm.at[pl.ds(base, SB)],
                             dst_loc.at[pl.ds(half, SB)], sem_st)

        def wait_stage():
            pltpu.make_async_copy(src_hbm.at[pl.ds(0, SB)],
                                  src_loc.at[pl.ds(0, SB)], sem_st).wait()
            pltpu.make_async_copy(dst_hbm.at[pl.ds(0, SB)],
                                  dst_loc.at[pl.ds(0, SB)], sem_st).wait()

        def fire_gather(off, buf):
            pltpu.async_copy(h_hbm.at[src_loc.at[pl.ds(off, 32)]],
                             rows3.at[buf], sem_g[buf])

        def drain_gather(buf):
            pltpu.make_async_copy(h_hbm.at[pl.ds(0, 32)],
                                  rows3.at[buf], sem_g[buf]).wait()

        def drain_scat(buf):
            pltpu.make_async_copy(h_hbm.at[pl.ds(0, 32)],
                                  rows3.at[buf], sem_s[buf]).wait()

        pltpu.sync_copy(src_hbm.at[pl.ds(b_base, SB)], src_loc.at[pl.ds(0, SB)])
        pltpu.sync_copy(dst_hbm.at[pl.ds(b_base, SB)], dst_loc.at[pl.ds(0, SB)])
        fire_gather(0, 0)

        def pb_it(it, carry):
            hoff = lax.rem(it, 2) * SB

            def wv_loop(i, carry2):
                e, r, lane = edge_vals(hoff // LANES + i)
                den = plsc.load_gather(den_loc, [r, lane])
                w_loc[pl.ds(i * LANES, LANES)] = e / (den + 1e-16)
                return carry2
            lax.fori_loop(0, SB // LANES, wv_loop, 0)

            for k in range(6):
                bcur = k % 3
                bnx = (k + 1) % 3
                t = it * 6 + k

                @pl.when(t >= 2)
                def _():
                    drain_scat(bnx)

                if k < 5:
                    fire_gather(hoff + (k + 1) * 32, bnx)
                else:
                    @pl.when(it < n_it - 1)
                    def _():
                        fire_gather((SB - hoff), bnx)

                if k == 0:
                    @pl.when(it < n_it - 1)
                    def _():
                        stage(it + 1)
                if k == 4:
                    @pl.when(it < n_it - 1)
                    def _():
                        wait_stage()

                drain_gather(bcur)

                def scale(g, carry2):
                    wv16 = w_loc[pl.ds(k * 32 + g * LANES, LANES)]
                    for kk in range(LANES):
                        wv = wv16[kk]
                        for j in range(d // LANES):
                            sl = pl.ds(j * LANES, LANES)
                            rows3[bcur, g * LANES + kk, sl] = (
                                rows3[bcur, g * LANES + kk, sl] * wv)
                    return carry2
                lax.fori_loop(0, 2, scale, 0)

                for k16 in range(2):
                    dvec = dst_loc[pl.ds(hoff + k * 32 + k16 * LANES, LANES)]
                    pltpu.async_copy(rows3.at[bcur, pl.ds(k16 * LANES, LANES)],
                                     out_acc.at[dvec], sem_s[bcur], add=True)
            return carry
        lax.fori_loop(0, n_it, pb_it, 0)

        drain_scat((6 * n_it - 2) % 3)
        drain_scat((6 * n_it - 1) % 3)

        plsc.subcore_barrier()
        pltpu.sync_copy(out_acc.at[pl.ds(s * zrows, zrows)],
                        out_hbm.at[c, pl.ds(s * zrows, zrows)])

    return sc_kernel


def kernel(x, edge_index, W, b, att):
    n, d = x.shape
    e0 = edge_index.shape[1]
    e1 = e0 + n

    n_pad = 10240
    n_acc = 10112
    step = N_SUB * SA
    e_pad = ((e1 + step - 1) // step) * step

    loops = jnp.arange(n, dtype=edge_index.dtype)
    ei = jnp.concatenate([edge_index, jnp.stack([loops, loops])], axis=1)
    pad = jnp.full((e_pad - e1,), n, dtype=jnp.int32)
    src = jnp.concatenate([ei[0], pad])
    dst = jnp.concatenate([ei[1], pad])

    x_p = jnp.concatenate([x, jnp.zeros((n_pad - n, d), x.dtype)])
    h, a2 = _linear_and_scores(x_p, W, b, att, n_pad, d)
    a = a2.reshape(n_pad)

    partials = _make_sc_kernel(n_pad, n_acc, d, e_pad)(h, a, src, dst)
    out = _combine(partials, n_acc, d)
    return (out[:n], edge_index)


# speedup vs baseline: 11.0385x; 1.0005x over previous
import functools

import jax
import jax.numpy as jnp
from jax import lax
from jax.experimental import pallas as pl
from jax.experimental.pallas import tpu as pltpu
from jax.experimental.pallas import tpu_sc as plsc

N_SC = 2
N_SUB = 16
LANES = 16


def _lin_body(x_ref, w_ref, b_ref, att_ref, h_ref, a_ref):
    x = x_ref[...]
    h = lax.dot_general(x, w_ref[...], (((1,), (1,)), ((), ())),
                        preferred_element_type=jnp.float32)
    h = h + b_ref[...]
    h_ref[...] = h
    a_ref[...] = jnp.sum(h * att_ref[...], axis=1, keepdims=True)


def _linear_and_scores(x_p, W, b, att, n_pad, d):
    rb = 1280
    grid = (n_pad // rb,)
    return pl.pallas_call(
        _lin_body,
        grid=grid,
        in_specs=[
            pl.BlockSpec((rb, d), lambda i: (i, 0)),
            pl.BlockSpec((d, d), lambda i: (0, 0)),
            pl.BlockSpec((1, d), lambda i: (0, 0)),
            pl.BlockSpec((1, d), lambda i: (0, 0)),
        ],
        out_specs=[
            pl.BlockSpec((rb, d), lambda i: (i, 0)),
            pl.BlockSpec((rb, 1), lambda i: (i, 0)),
        ],
        out_shape=[
            jax.ShapeDtypeStruct((n_pad, d), jnp.float32),
            jax.ShapeDtypeStruct((n_pad, 1), jnp.float32),
        ],
    )(x_p, W, b.reshape(1, d), att.reshape(1, d))


def _fin_body(p_ref, o_ref):
    o_ref[...] = jnp.maximum(p_ref[0] + p_ref[1], 0.0)


def _combine(partials, n_acc, d):
    rb = n_acc // 8
    return pl.pallas_call(
        _fin_body,
        grid=(n_acc // rb,),
        in_specs=[pl.BlockSpec((2, rb, d), lambda i: (0, i, 0))],
        out_specs=pl.BlockSpec((rb, d), lambda i: (i, 0)),
        out_shape=jax.ShapeDtypeStruct((n_acc, d), jnp.float32),
    )(partials)


SA = 3072
SB = 192


def _make_sc_kernel(n_pad, n_acc, d, e_pad):
    ch_a = e_pad // N_SUB
    ch_b = e_pad // (N_SUB * N_SC)
    den_rows = n_pad // 128
    zrows = n_acc // N_SUB
    mesh = plsc.VectorSubcoreMesh(core_axis_name="c", subcore_axis_name="s")

    @functools.partial(
        pl.kernel,
        out_type=jax.ShapeDtypeStruct((N_SC, n_acc, d), jnp.float32),
        mesh=mesh,
        compiler_params=pltpu.CompilerParams(needs_layout_passes=False),
        scratch_types=[
            pltpu.VMEM((n_pad,), jnp.float32),
            pltpu.VMEM((den_rows, 128), jnp.float32),
            pltpu.VMEM((SA,), jnp.int32),
            pltpu.VMEM((SA,), jnp.int32),
            pltpu.VMEM((SB,), jnp.float32),
            pltpu.VMEM((3, 32, d), jnp.float32),
            pltpu.VMEM_SHARED((n_acc, d), jnp.float32),
            pltpu.VMEM_SHARED((den_rows, 128), jnp.float32),
            pltpu.SemaphoreType.DMA,
            pltpu.SemaphoreType.DMA,
            pltpu.SemaphoreType.DMA,
            pltpu.SemaphoreType.DMA,
            pltpu.SemaphoreType.DMA,
            pltpu.SemaphoreType.DMA,
            pltpu.SemaphoreType.DMA,
        ],
    )
    def sc_kernel(h_hbm, a_hbm, src_hbm, dst_hbm, out_hbm,
                  a_loc, den_loc, src_loc, dst_loc, w_loc, rows3,
                  out_acc, den_sh, sem_st, sem_g0, sem_g1, sem_g2,
                  sem_s0, sem_s1, sem_s2):
        sem_g = (sem_g0, sem_g1, sem_g2)
        sem_s = (sem_s0, sem_s1, sem_s2)
        c = lax.axis_index("c")
        s = lax.axis_index("s")
        z16 = jnp.zeros((LANES,), jnp.float32)

        pltpu.sync_copy(a_hbm, a_loc)

        def zden(i, carry):
            for j in range(128 // LANES):
                den_loc[i, pl.ds(j * LANES, LANES)] = z16
            return carry
        lax.fori_loop(0, den_rows, zden, 0)

        def zrow(i, carry):
            for bb in range(3):
                for j in range(d // LANES):
                    rows3[bb, i, pl.ds(j * LANES, LANES)] = z16
            return carry
        lax.fori_loop(0, 32, zrow, 0)

        @pl.when(s == 0)
        def _():
            pltpu.sync_copy(den_loc, den_sh)

        for k in range(zrows // 32):
            pltpu.sync_copy(rows3.at[0], out_acc.at[pl.ds(s * zrows + k * 32, 32)])
        rem = zrows % 32
        if rem:
            pltpu.sync_copy(rows3.at[0, pl.ds(0, rem)],
                            out_acc.at[pl.ds(s * zrows + zrows - rem, rem)])

        def edge_vals(i):
            si = src_loc[pl.ds(i * LANES, LANES)]
            di = dst_loc[pl.ds(i * LANES, LANES)]
            av = plsc.load_gather(a_loc, [si]) + plsc.load_gather(a_loc, [di])
            al = jnp.where(av >= 0.0, av, 0.2 * av)
            e = jnp.exp(al)
            r = lax.shift_right_logical(si, 7)
            lane = jnp.bitwise_and(si, 127)
            return e, r, lane

        sc_a = jax.named_scope("ph_passA"); sc_a.__enter__()

        def pa_blk(t, carry):
            base = s * ch_a + t * SA
            pltpu.sync_copy(src_hbm.at[pl.ds(base, SA)], src_loc)
            pltpu.sync_copy(dst_hbm.at[pl.ds(base, SA)], dst_loc)

            def pa(i, carry2):
                e, r, lane = edge_vals(i)
                plsc.addupdate_scatter(den_loc, [r, lane], e)
                return carry2
            lax.fori_loop(0, SA // LANES, pa, 0)
            return carry
        lax.fori_loop(0, ch_a // SA, pa_blk, 0)
        sc_a.__exit__(None, None, None)

        sc_r = jax.named_scope("ph_denred"); sc_r.__enter__()
        plsc.subcore_barrier()
        for k in range(den_rows // LANES):
            ivec = lax.iota(jnp.int32, LANES) + k * LANES
            pltpu.sync_copy(den_loc.at[pl.ds(k * LANES, LANES)],
                            den_sh.at[ivec], add=True)
        plsc.subcore_barrier()
        pltpu.sync_copy(den_sh, den_loc)
        sc_r.__exit__(None, None, None)
        sc_b = jax.named_scope("ph_passB"); sc_b.__enter__()

        wid = s * N_SC + c
        n_it = ch_b // SB
        b_base = wid * ch_b

        def stage(it):
            half = lax.rem(it, 2) * SB
            base = b_base + it * SB
            pltpu.async_copy(src_hbm.at[pl.ds(base, SB)],
                             src_loc.at[pl.ds(half, SB)], sem_st)
            pltpu.async_copy(dst_hbm.at[pl.ds(base, SB)],
                             dst_loc.at[pl.ds(half, SB)], sem_st)

        def wait_stage():
            pltpu.make_async_copy(src_hbm.at[pl.ds(0, SB)],
                                  src_loc.at[pl.ds(0, SB)], sem_st).wait()
            pltpu.make_async_copy(dst_hbm.at[pl.ds(0, SB)],
                                  dst_loc.at[pl.ds(0, SB)], sem_st).wait()

        def fire_gather(off, buf):
            pltpu.async_copy(h_hbm.at[src_loc.at[pl.ds(off, 32)]],
                             rows3.at[buf], sem_g[buf])

        def drain_gather(buf):
            pltpu.make_async_copy(h_hbm.at[pl.ds(0, 32)],
                                  rows3.at[buf], sem_g[buf]).wait()

        def drain_scat(buf):
            pltpu.make_async_copy(h_hbm.at[pl.ds(0, 32)],
                                  rows3.at[buf], sem_s[buf]).wait()

        pltpu.sync_copy(src_hbm.at[pl.ds(b_base, SB)], src_loc.at[pl.ds(0, SB)])
        pltpu.sync_copy(dst_hbm.at[pl.ds(b_base, SB)], dst_loc.at[pl.ds(0, SB)])
        fire_gather(0, 0)

        def pb_it(it, carry):
            hoff = lax.rem(it, 2) * SB

            def wv_loop(i, carry2):
                e, r, lane = edge_vals(hoff // LANES + i)
                den = plsc.load_gather(den_loc, [r, lane])
                w_loc[pl.ds(i * LANES, LANES)] = e / (den + 1e-16)
                return carry2
            lax.fori_loop(0, SB // LANES, wv_loop, 0)

            for k in range(6):
                bcur = k % 3
                bnx = (k + 1) % 3
                t = it * 6 + k

                @pl.when(t >= 2)
                def _():
                    drain_scat(bnx)

                if k < 5:
                    fire_gather(hoff + (k + 1) * 32, bnx)
                else:
                    @pl.when(it < n_it - 1)
                    def _():
                        fire_gather((SB - hoff), bnx)

                if k == 0:
                    @pl.when(it < n_it - 1)
                    def _():
                        stage(it + 1)
                if k == 4:
                    @pl.when(it < n_it - 1)
                    def _():
                        wait_stage()

                drain_gather(bcur)

                def scale(g, carry2):
                    wv16 = w_loc[pl.ds(k * 32 + g * LANES, LANES)]
                    for kk in range(LANES):
                        wv = wv16[kk]
                        for j in range(d // LANES):
                            sl = pl.ds(j * LANES, LANES)
                            rows3[bcur, g * LANES + kk, sl] = (
                                rows3[bcur, g * LANES + kk, sl] * wv)
                    return carry2
                lax.fori_loop(0, 2, scale, 0)

                for k16 in range(2):
                    dvec = dst_loc[pl.ds(hoff + k * 32 + k16 * LANES, LANES)]
                    pltpu.async_copy(rows3.at[bcur, pl.ds(k16 * LANES, LANES)],
                                     out_acc.at[dvec], sem_s[bcur], add=True)
            return carry
        lax.fori_loop(0, n_it, pb_it, 0)

        drain_scat((6 * n_it - 2) % 3)
        drain_scat((6 * n_it - 1) % 3)
        sc_b.__exit__(None, None, None)

        plsc.subcore_barrier()
        pltpu.sync_copy(out_acc.at[pl.ds(s * zrows, zrows)],
                        out_hbm.at[c, pl.ds(s * zrows, zrows)])

    return sc_kernel


def kernel(x, edge_index, W, b, att):
    n, d = x.shape
    e0 = edge_index.shape[1]
    e1 = e0 + n

    n_pad = 10240
    n_acc = 10112
    step = N_SUB * SA
    e_pad = ((e1 + step - 1) // step) * step

    loops = jnp.arange(n, dtype=edge_index.dtype)
    ei = jnp.concatenate([edge_index, jnp.stack([loops, loops])], axis=1)
    pad = jnp.full((e_pad - e1,), n, dtype=jnp.int32)
    src = jnp.concatenate([ei[0], pad])
    dst = jnp.concatenate([ei[1], pad])

    x_p = jnp.concatenate([x, jnp.zeros((n_pad - n, d), x.dtype)])
    h, a2 = _linear_and_scores(x_p, W, b, att, n_pad, d)
    a = a2.reshape(n_pad)

    partials = _make_sc_kernel(n_pad, n_acc, d, e_pad)(h, a, src, dst)
    out = _combine(partials, n_acc, d)
    return (out[:n], edge_index)


# speedup vs baseline: 29.9243x; 2.7109x over previous
import functools

import jax
import jax.numpy as jnp
from jax import lax
from jax.experimental import pallas as pl
from jax.experimental.pallas import tpu as pltpu
from jax.experimental.pallas import tpu_sc as plsc

N_SC = 2
N_SUB = 16
LANES = 16


def _lin_body(x_ref, w_ref, b_ref, att_ref, h_ref, a_ref):
    x = x_ref[...]
    h = lax.dot_general(x, w_ref[...], (((1,), (1,)), ((), ())),
                        preferred_element_type=jnp.float32)
    h = h + b_ref[...]
    h_ref[...] = h
    a_ref[...] = jnp.sum(h * att_ref[...], axis=1, keepdims=True)


def _linear_and_scores(x_p, W, b, att, n_pad, d):
    rb = 1280
    grid = (n_pad // rb,)
    return pl.pallas_call(
        _lin_body,
        grid=grid,
        in_specs=[
            pl.BlockSpec((rb, d), lambda i: (i, 0)),
            pl.BlockSpec((d, d), lambda i: (0, 0)),
            pl.BlockSpec((1, d), lambda i: (0, 0)),
            pl.BlockSpec((1, d), lambda i: (0, 0)),
        ],
        out_specs=[
            pl.BlockSpec((rb, d), lambda i: (i, 0)),
            pl.BlockSpec((rb, 1), lambda i: (i, 0)),
        ],
        out_shape=[
            jax.ShapeDtypeStruct((n_pad, d), jnp.float32),
            jax.ShapeDtypeStruct((n_pad, 1), jnp.float32),
        ],
    )(x_p, W, b.reshape(1, d), att.reshape(1, d))


def _fin_body(p_ref, o_ref):
    o_ref[...] = jnp.maximum(p_ref[0] + p_ref[1], 0.0)


def _combine(partials, n_acc, d):
    rb = n_acc // 8
    return pl.pallas_call(
        _fin_body,
        grid=(n_acc // rb,),
        in_specs=[pl.BlockSpec((2, rb, d), lambda i: (0, i, 0))],
        out_specs=pl.BlockSpec((rb, d), lambda i: (i, 0)),
        out_shape=jax.ShapeDtypeStruct((n_acc, d), jnp.float32),
    )(partials)


SA = 1296
SB = 192


def _make_sc_kernel(n_pad, n_acc, d, e_pad):
    ch_a = e_pad // N_SUB
    ch_b = e_pad // (N_SUB * N_SC)
    den_rows = n_pad // 128
    zrows = n_acc // N_SUB
    mesh = plsc.VectorSubcoreMesh(core_axis_name="c", subcore_axis_name="s")

    @functools.partial(
        pl.kernel,
        out_type=jax.ShapeDtypeStruct((N_SC, n_acc, d), jnp.float32),
        mesh=mesh,
        compiler_params=pltpu.CompilerParams(needs_layout_passes=False),
        scratch_types=[
            pltpu.VMEM((n_pad,), jnp.float32),
            pltpu.VMEM((den_rows, 128), jnp.float32),
            pltpu.VMEM((2 * SA,), jnp.int32),
            pltpu.VMEM((2 * SA,), jnp.int32),
            pltpu.VMEM((SB,), jnp.float32),
            pltpu.VMEM((3, 32, d), jnp.float32),
            pltpu.VMEM_SHARED((n_acc, d), jnp.float32),
            pltpu.VMEM_SHARED((den_rows, 128), jnp.float32),
            pltpu.SemaphoreType.DMA,
            pltpu.SemaphoreType.DMA,
            pltpu.SemaphoreType.DMA,
            pltpu.SemaphoreType.DMA,
            pltpu.SemaphoreType.DMA,
            pltpu.SemaphoreType.DMA,
            pltpu.SemaphoreType.DMA,
        ],
    )
    def sc_kernel(h_hbm, a_hbm, src_hbm, dst_hbm, out_hbm,
                  a_loc, den_loc, src_loc, dst_loc, w_loc, rows3,
                  out_acc, den_sh, sem_st, sem_g0, sem_g1, sem_g2,
                  sem_s0, sem_s1, sem_s2):
        sem_g = (sem_g0, sem_g1, sem_g2)
        sem_s = (sem_s0, sem_s1, sem_s2)
        c = lax.axis_index("c")
        s = lax.axis_index("s")
        z16 = jnp.zeros((LANES,), jnp.float32)

        pltpu.sync_copy(a_hbm, a_loc)

        def zden(i, carry):
            for j in range(128 // LANES):
                den_loc[i, pl.ds(j * LANES, LANES)] = z16
            return carry
        lax.fori_loop(0, den_rows, zden, 0)

        def zrow(i, carry):
            for bb in range(3):
                for j in range(d // LANES):
                    rows3[bb, i, pl.ds(j * LANES, LANES)] = z16
            return carry
        lax.fori_loop(0, 32, zrow, 0)

        @pl.when(s == 0)
        def _():
            pltpu.sync_copy(den_loc, den_sh)

        for k in range(zrows // 32):
            pltpu.sync_copy(rows3.at[0], out_acc.at[pl.ds(s * zrows + k * 32, 32)])
        rem = zrows % 32
        if rem:
            pltpu.sync_copy(rows3.at[0, pl.ds(0, rem)],
                            out_acc.at[pl.ds(s * zrows + zrows - rem, rem)])

        def edge_vals(i):
            si = src_loc[pl.ds(i * LANES, LANES)]
            di = dst_loc[pl.ds(i * LANES, LANES)]
            av = plsc.load_gather(a_loc, [si]) + plsc.load_gather(a_loc, [di])
            al = jnp.where(av >= 0.0, av, 0.2 * av)
            e = jnp.exp(al)
            r = lax.shift_right_logical(si, 7)
            lane = jnp.bitwise_and(si, 127)
            return e, r, lane

        sc_a = jax.named_scope("ph_passA"); sc_a.__enter__()
        n_blk_a = ch_a // SA

        def stage_a(t):
            half = lax.rem(t, 2) * SA
            base = s * ch_a + t * SA
            pltpu.async_copy(src_hbm.at[pl.ds(base, SA)],
                             src_loc.at[pl.ds(half, SA)], sem_st)
            pltpu.async_copy(dst_hbm.at[pl.ds(base, SA)],
                             dst_loc.at[pl.ds(half, SA)], sem_st)

        def wait_stage_a():
            pltpu.make_async_copy(src_hbm.at[pl.ds(0, SA)],
                                  src_loc.at[pl.ds(0, SA)], sem_st).wait()
            pltpu.make_async_copy(dst_hbm.at[pl.ds(0, SA)],
                                  dst_loc.at[pl.ds(0, SA)], sem_st).wait()

        stage_a(0)

        def pa_blk(t, carry):
            wait_stage_a()

            @pl.when(t < n_blk_a - 1)
            def _():
                stage_a(t + 1)
            hoff_a = lax.rem(t, 2) * SA

            def pa(i, carry2):
                e, r, lane = edge_vals(hoff_a // LANES + i)
                plsc.addupdate_scatter(den_loc, [r, lane], e)
                return carry2
            lax.fori_loop(0, SA // LANES, pa, 0)
            return carry
        lax.fori_loop(0, n_blk_a, pa_blk, 0)
        sc_a.__exit__(None, None, None)

        sc_r = jax.named_scope("ph_denred"); sc_r.__enter__()
        plsc.subcore_barrier()
        for k in range(den_rows // LANES):
            ivec = lax.iota(jnp.int32, LANES) + k * LANES
            pltpu.sync_copy(den_loc.at[pl.ds(k * LANES, LANES)],
                            den_sh.at[ivec], add=True)
        plsc.subcore_barrier()
        pltpu.sync_copy(den_sh, den_loc)
        sc_r.__exit__(None, None, None)
        sc_b = jax.named_scope("ph_passB"); sc_b.__enter__()

        wid = s * N_SC + c
        n_it = ch_b // SB
        b_base = wid * ch_b

        def stage(it):
            half = lax.rem(it, 2) * SB
            base = b_base + it * SB
            pltpu.async_copy(src_hbm.at[pl.ds(base, SB)],
                             src_loc.at[pl.ds(half, SB)], sem_st)
            pltpu.async_copy(dst_hbm.at[pl.ds(base, SB)],
                             dst_loc.at[pl.ds(half, SB)], sem_st)

        def wait_stage():
            pltpu.make_async_copy(src_hbm.at[pl.ds(0, SB)],
                                  src_loc.at[pl.ds(0, SB)], sem_st).wait()
            pltpu.make_async_copy(dst_hbm.at[pl.ds(0, SB)],
                                  dst_loc.at[pl.ds(0, SB)], sem_st).wait()

        def fire_gather(off, buf):
            pltpu.async_copy(h_hbm.at[src_loc.at[pl.ds(off, 32)]],
                             rows3.at[buf], sem_g[buf])

        def drain_gather(buf):
            pltpu.make_async_copy(h_hbm.at[pl.ds(0, 32)],
                                  rows3.at[buf], sem_g[buf]).wait()

        def drain_scat(buf):
            pltpu.make_async_copy(h_hbm.at[pl.ds(0, 32)],
                                  rows3.at[buf], sem_s[buf]).wait()

        pltpu.sync_copy(src_hbm.at[pl.ds(b_base, SB)], src_loc.at[pl.ds(0, SB)])
        pltpu.sync_copy(dst_hbm.at[pl.ds(b_base, SB)], dst_loc.at[pl.ds(0, SB)])
        fire_gather(0, 0)

        def pb_it(it, carry):
            hoff = lax.rem(it, 2) * SB

            def wv_loop(i, carry2):
                e, r, lane = edge_vals(hoff // LANES + i)
                den = plsc.load_gather(den_loc, [r, lane])
                w_loc[pl.ds(i * LANES, LANES)] = e / (den + 1e-16)
                return carry2
            lax.fori_loop(0, SB // LANES, wv_loop, 0)

            for k in range(6):
                bcur = k % 3
                bnx = (k + 1) % 3
                t = it * 6 + k

                @pl.when(t >= 2)
                def _():
                    drain_scat(bnx)

                if k < 5:
                    fire_gather(hoff + (k + 1) * 32, bnx)
                else:
                    @pl.when(it < n_it - 1)
                    def _():
                        fire_gather((SB - hoff), bnx)

                if k == 0:
                    @pl.when(it < n_it - 1)
                    def _():
                        stage(it + 1)
                if k == 4:
                    @pl.when(it < n_it - 1)
                    def _():
                        wait_stage()

                drain_gather(bcur)

                def scale(g, carry2):
                    wv16 = w_loc[pl.ds(k * 32 + g * LANES, LANES)]
                    for kk in range(LANES):
                        wv = wv16[kk]
                        for j in range(d // LANES):
                            sl = pl.ds(j * LANES, LANES)
                            rows3[bcur, g * LANES + kk, sl] = (
                                rows3[bcur, g * LANES + kk, sl] * wv)
                    return carry2
                lax.fori_loop(0, 2, scale, 0)

                for k16 in range(2):
                    dvec = dst_loc[pl.ds(hoff + k * 32 + k16 * LANES, LANES)]
                    pltpu.async_copy(rows3.at[bcur, pl.ds(k16 * LANES, LANES)],
                                     out_acc.at[dvec], sem_s[bcur], add=True)
            return carry
        lax.fori_loop(0, n_it, pb_it, 0)

        drain_scat((6 * n_it - 2) % 3)
        drain_scat((6 * n_it - 1) % 3)
        sc_b.__exit__(None, None, None)

        plsc.subcore_barrier()
        pltpu.sync_copy(out_acc.at[pl.ds(s * zrows, zrows)],
                        out_hbm.at[c, pl.ds(s * zrows, zrows)])

    return sc_kernel


def kernel(x, edge_index, W, b, att):
    n, d = x.shape
    e0 = edge_index.shape[1]
    e1 = e0 + n

    n_pad = 10240
    n_acc = 10112
    step = 165888
    e_pad = ((e1 + step - 1) // step) * step

    loops = jnp.arange(n, dtype=edge_index.dtype)
    ei = jnp.concatenate([edge_index, jnp.stack([loops, loops])], axis=1)
    pad = n + jnp.arange(e_pad - e1, dtype=jnp.int32) % (n_acc - n)
    src = jnp.concatenate([ei[0], pad])
    dst = jnp.concatenate([ei[1], pad])

    x_p = jnp.concatenate([x, jnp.zeros((n_pad - n, d), x.dtype)])
    h, a2 = _linear_and_scores(x_p, W, b, att, n_pad, d)
    a = a2.reshape(n_pad)

    partials = _make_sc_kernel(n_pad, n_acc, d, e_pad)(h, a, src, dst)
    out = _combine(partials, n_acc, d)
    return (out[:n], edge_index)


# speedup vs baseline: 30.8016x; 1.0293x over previous
import functools

import jax
import jax.numpy as jnp
from jax import lax
from jax.experimental import pallas as pl
from jax.experimental.pallas import tpu as pltpu
from jax.experimental.pallas import tpu_sc as plsc

N_SC = 2
N_SUB = 16
LANES = 16


def _lin_body(x_ref, w_ref, b_ref, att_ref, h_ref, a_ref):
    x = x_ref[...]
    h = lax.dot_general(x, w_ref[...], (((1,), (1,)), ((), ())),
                        preferred_element_type=jnp.float32)
    h = h + b_ref[...]
    h_ref[...] = h
    a_ref[...] = jnp.sum(h * att_ref[...], axis=1, keepdims=True)


def _linear_and_scores(x_p, W, b, att, n_pad, d):
    rb = 1280
    grid = (n_pad // rb,)
    return pl.pallas_call(
        _lin_body,
        grid=grid,
        in_specs=[
            pl.BlockSpec((rb, d), lambda i: (i, 0)),
            pl.BlockSpec((d, d), lambda i: (0, 0)),
            pl.BlockSpec((1, d), lambda i: (0, 0)),
            pl.BlockSpec((1, d), lambda i: (0, 0)),
        ],
        out_specs=[
            pl.BlockSpec((rb, d), lambda i: (i, 0)),
            pl.BlockSpec((rb, 1), lambda i: (i, 0)),
        ],
        out_shape=[
            jax.ShapeDtypeStruct((n_pad, d), jnp.float32),
            jax.ShapeDtypeStruct((n_pad, 1), jnp.float32),
        ],
    )(x_p, W, b.reshape(1, d), att.reshape(1, d))


def _fin_body(p_ref, o_ref):
    o_ref[...] = jnp.maximum(p_ref[0] + p_ref[1], 0.0)


def _combine(partials, n_acc, d):
    rb = n_acc // 8
    return pl.pallas_call(
        _fin_body,
        grid=(n_acc // rb,),
        in_specs=[pl.BlockSpec((2, rb, d), lambda i: (0, i, 0))],
        out_specs=pl.BlockSpec((rb, d), lambda i: (i, 0)),
        out_shape=jax.ShapeDtypeStruct((n_acc, d), jnp.float32),
    )(partials)


SA = 1296
SB = 288


def _make_sc_kernel(n_pad, n_acc, d, e_pad):
    ch_a = e_pad // N_SUB
    ch_b = e_pad // (N_SUB * N_SC)
    den_rows = n_pad // 128
    zrows = n_acc // N_SUB
    mesh = plsc.VectorSubcoreMesh(core_axis_name="c", subcore_axis_name="s")

    @functools.partial(
        pl.kernel,
        out_type=jax.ShapeDtypeStruct((N_SC, n_acc, d), jnp.float32),
        mesh=mesh,
        compiler_params=pltpu.CompilerParams(needs_layout_passes=False),
        scratch_types=[
            pltpu.VMEM((n_pad,), jnp.float32),
            pltpu.VMEM((den_rows, 128), jnp.float32),
            pltpu.VMEM((2 * SA,), jnp.int32),
            pltpu.VMEM((2 * SA,), jnp.int32),
            pltpu.VMEM((SB,), jnp.float32),
            pltpu.VMEM((3, 48, d), jnp.float32),
            pltpu.VMEM_SHARED((n_acc, d), jnp.float32),
            pltpu.VMEM_SHARED((den_rows, 128), jnp.float32),
            pltpu.SemaphoreType.DMA,
            pltpu.SemaphoreType.DMA,
            pltpu.SemaphoreType.DMA,
            pltpu.SemaphoreType.DMA,
            pltpu.SemaphoreType.DMA,
            pltpu.SemaphoreType.DMA,
            pltpu.SemaphoreType.DMA,
        ],
    )
    def sc_kernel(h_hbm, a_hbm, src_hbm, dst_hbm, out_hbm,
                  a_loc, den_loc, src_loc, dst_loc, w_loc, rows3,
                  out_acc, den_sh, sem_st, sem_g0, sem_g1, sem_g2,
                  sem_s0, sem_s1, sem_s2):
        sem_g = (sem_g0, sem_g1, sem_g2)
        sem_s = (sem_s0, sem_s1, sem_s2)
        c = lax.axis_index("c")
        s = lax.axis_index("s")
        z16 = jnp.zeros((LANES,), jnp.float32)

        pltpu.sync_copy(a_hbm, a_loc)

        def zden(i, carry):
            for j in range(128 // LANES):
                den_loc[i, pl.ds(j * LANES, LANES)] = z16
            return carry
        lax.fori_loop(0, den_rows, zden, 0)

        def zrow(i, carry):
            for bb in range(3):
                for j in range(d // LANES):
                    rows3[bb, i, pl.ds(j * LANES, LANES)] = z16
            return carry
        lax.fori_loop(0, 48, zrow, 0)

        @pl.when(s == 0)
        def _():
            pltpu.sync_copy(den_loc, den_sh)

        for k in range(zrows // 48):
            pltpu.sync_copy(rows3.at[0], out_acc.at[pl.ds(s * zrows + k * 48, 48)])
        rem = zrows % 48
        if rem:
            pltpu.sync_copy(rows3.at[0, pl.ds(0, rem)],
                            out_acc.at[pl.ds(s * zrows + zrows - rem, rem)])

        def edge_vals(i):
            si = src_loc[pl.ds(i * LANES, LANES)]
            di = dst_loc[pl.ds(i * LANES, LANES)]
            av = plsc.load_gather(a_loc, [si]) + plsc.load_gather(a_loc, [di])
            al = jnp.where(av >= 0.0, av, 0.2 * av)
            e = jnp.exp(al)
            r = lax.shift_right_logical(si, 7)
            lane = jnp.bitwise_and(si, 127)
            return e, r, lane

        sc_a = jax.named_scope("ph_passA"); sc_a.__enter__()
        n_blk_a = ch_a // SA

        def stage_a(t):
            half = lax.rem(t, 2) * SA
            base = s * ch_a + t * SA
            pltpu.async_copy(src_hbm.at[pl.ds(base, SA)],
                             src_loc.at[pl.ds(half, SA)], sem_st)
            pltpu.async_copy(dst_hbm.at[pl.ds(base, SA)],
                             dst_loc.at[pl.ds(half, SA)], sem_st)

        def wait_stage_a():
            pltpu.make_async_copy(src_hbm.at[pl.ds(0, SA)],
                                  src_loc.at[pl.ds(0, SA)], sem_st).wait()
            pltpu.make_async_copy(dst_hbm.at[pl.ds(0, SA)],
                                  dst_loc.at[pl.ds(0, SA)], sem_st).wait()

        stage_a(0)

        def pa_blk(t, carry):
            wait_stage_a()

            @pl.when(t < n_blk_a - 1)
            def _():
                stage_a(t + 1)
            hoff_a = lax.rem(t, 2) * SA

            def pa(i, carry2):
                e, r, lane = edge_vals(hoff_a // LANES + i)
                plsc.addupdate_scatter(den_loc, [r, lane], e)
                return carry2
            lax.fori_loop(0, SA // LANES, pa, 0, unroll=2)
            return carry
        lax.fori_loop(0, n_blk_a, pa_blk, 0)
        sc_a.__exit__(None, None, None)

        sc_r = jax.named_scope("ph_denred"); sc_r.__enter__()
        plsc.subcore_barrier()
        for k in range(den_rows // LANES):
            ivec = lax.iota(jnp.int32, LANES) + k * LANES
            pltpu.sync_copy(den_loc.at[pl.ds(k * LANES, LANES)],
                            den_sh.at[ivec], add=True)
        plsc.subcore_barrier()
        pltpu.sync_copy(den_sh, den_loc)
        sc_r.__exit__(None, None, None)
        sc_b = jax.named_scope("ph_passB"); sc_b.__enter__()

        wid = s * N_SC + c
        n_it = ch_b // SB
        b_base = wid * ch_b

        def stage(it):
            half = lax.rem(it, 2) * SB
            base = b_base + it * SB
            pltpu.async_copy(src_hbm.at[pl.ds(base, SB)],
                             src_loc.at[pl.ds(half, SB)], sem_st)
            pltpu.async_copy(dst_hbm.at[pl.ds(base, SB)],
                             dst_loc.at[pl.ds(half, SB)], sem_st)

        def wait_stage():
            pltpu.make_async_copy(src_hbm.at[pl.ds(0, SB)],
                                  src_loc.at[pl.ds(0, SB)], sem_st).wait()
            pltpu.make_async_copy(dst_hbm.at[pl.ds(0, SB)],
                                  dst_loc.at[pl.ds(0, SB)], sem_st).wait()

        def fire_gather(off, buf):
            pltpu.async_copy(h_hbm.at[src_loc.at[pl.ds(off, 48)]],
                             rows3.at[buf], sem_g[buf])

        def drain_gather(buf):
            pltpu.make_async_copy(h_hbm.at[pl.ds(0, 48)],
                                  rows3.at[buf], sem_g[buf]).wait()

        def drain_scat(buf):
            pltpu.make_async_copy(h_hbm.at[pl.ds(0, 48)],
                                  rows3.at[buf], sem_s[buf]).wait()

        pltpu.sync_copy(src_hbm.at[pl.ds(b_base, SB)], src_loc.at[pl.ds(0, SB)])
        pltpu.sync_copy(dst_hbm.at[pl.ds(b_base, SB)], dst_loc.at[pl.ds(0, SB)])
        fire_gather(0, 0)

        def pb_it(it, carry):
            hoff = lax.rem(it, 2) * SB

            def wv_loop(i, carry2):
                e, r, lane = edge_vals(hoff // LANES + i)
                den = plsc.load_gather(den_loc, [r, lane])
                w_loc[pl.ds(i * LANES, LANES)] = e / (den + 1e-16)
                return carry2
            lax.fori_loop(0, SB // LANES, wv_loop, 0)

            for k in range(6):
                bcur = k % 3
                bnx = (k + 1) % 3
                t = it * 6 + k

                @pl.when(t >= 2)
                def _():
                    drain_scat(bnx)

                if k < 5:
                    fire_gather(hoff + (k + 1) * 48, bnx)
                else:
                    @pl.when(it < n_it - 1)
                    def _():
                        fire_gather((SB - hoff), bnx)

                if k == 0:
                    @pl.when(it < n_it - 1)
                    def _():
                        stage(it + 1)
                if k == 4:
                    @pl.when(it < n_it - 1)
                    def _():
                        wait_stage()

                drain_gather(bcur)

                def scale(g, carry2):
                    wv16 = w_loc[pl.ds(k * 48 + g * LANES, LANES)]
                    for kk in range(LANES):
                        wv = wv16[kk]
                        for j in range(d // LANES):
                            sl = pl.ds(j * LANES, LANES)
                            rows3[bcur, g * LANES + kk, sl] = (
                                rows3[bcur, g * LANES + kk, sl] * wv)
                    return carry2
                lax.fori_loop(0, 3, scale, 0)

                for k16 in range(3):
                    dvec = dst_loc[pl.ds(hoff + k * 48 + k16 * LANES, LANES)]
                    pltpu.async_copy(rows3.at[bcur, pl.ds(k16 * LANES, LANES)],
                                     out_acc.at[dvec], sem_s[bcur], add=True)
            return carry
        lax.fori_loop(0, n_it, pb_it, 0)

        drain_scat((6 * n_it - 2) % 3)
        drain_scat((6 * n_it - 1) % 3)
        sc_b.__exit__(None, None, None)

        plsc.subcore_barrier()
        pltpu.sync_copy(out_acc.at[pl.ds(s * zrows, zrows)],
                        out_hbm.at[c, pl.ds(s * zrows, zrows)])

    return sc_kernel


def kernel(x, edge_index, W, b, att):
    n, d = x.shape
    e0 = edge_index.shape[1]
    e1 = e0 + n

    n_pad = 10240
    n_acc = 10112
    step = 82944
    e_pad = ((e1 + step - 1) // step) * step

    loops = jnp.arange(n, dtype=edge_index.dtype)
    ei = jnp.concatenate([edge_index, jnp.stack([loops, loops])], axis=1)
    pad = n + jnp.arange(e_pad - e1, dtype=jnp.int32) % (n_acc - n)
    src = jnp.concatenate([ei[0], pad])
    dst = jnp.concatenate([ei[1], pad])

    x_p = jnp.concatenate([x, jnp.zeros((n_pad - n, d), x.dtype)])
    h, a2 = _linear_and_scores(x_p, W, b, att, n_pad, d)
    a = a2.reshape(n_pad)

    partials = _make_sc_kernel(n_pad, n_acc, d, e_pad)(h, a, src, dst)
    out = _combine(partials, n_acc, d)
    return (out[:n], edge_index)


# speedup vs baseline: 36.7520x; 1.1932x over previous
import functools

import jax
import jax.numpy as jnp
from jax import lax
from jax.experimental import pallas as pl
from jax.experimental.pallas import tpu as pltpu
from jax.experimental.pallas import tpu_sc as plsc

N_SC = 2
N_SUB = 16
LANES = 16


def _lin_body(x_ref, w_ref, b_ref, att_ref, h_ref, a_ref):
    x = x_ref[...]
    h = lax.dot_general(x, w_ref[...], (((1,), (1,)), ((), ())),
                        preferred_element_type=jnp.float32)
    h = h + b_ref[...]
    h_ref[...] = h
    a_ref[...] = jnp.sum(h * att_ref[...], axis=1, keepdims=True)


def _linear_and_scores(x_p, W, b, att, n_pad, d):
    rb = 1280
    grid = (n_pad // rb,)
    return pl.pallas_call(
        _lin_body,
        grid=grid,
        in_specs=[
            pl.BlockSpec((rb, d), lambda i: (i, 0)),
            pl.BlockSpec((d, d), lambda i: (0, 0)),
            pl.BlockSpec((1, d), lambda i: (0, 0)),
            pl.BlockSpec((1, d), lambda i: (0, 0)),
        ],
        out_specs=[
            pl.BlockSpec((rb, d), lambda i: (i, 0)),
            pl.BlockSpec((rb, 1), lambda i: (i, 0)),
        ],
        out_shape=[
            jax.ShapeDtypeStruct((n_pad, d), jnp.float32),
            jax.ShapeDtypeStruct((n_pad, 1), jnp.float32),
        ],
    )(x_p, W, b.reshape(1, d), att.reshape(1, d))


def _fin_body(p_ref, o_ref):
    o_ref[...] = jnp.maximum(p_ref[0] + p_ref[1], 0.0)


def _combine(partials, n_acc, d):
    rb = n_acc // 8
    return pl.pallas_call(
        _fin_body,
        grid=(n_acc // rb,),
        in_specs=[pl.BlockSpec((2, rb, d), lambda i: (0, i, 0))],
        out_specs=pl.BlockSpec((rb, d), lambda i: (i, 0)),
        out_shape=jax.ShapeDtypeStruct((n_acc, d), jnp.float32),
    )(partials)


SA = 1296
SB = 128


def _make_sc_kernel(n_pad, n_acc, d, e_pad):
    ch_a = e_pad // N_SUB
    ch_b = e_pad // (N_SUB * N_SC)
    den_rows = n_pad // 128
    zrows = n_acc // N_SUB
    mesh = plsc.VectorSubcoreMesh(core_axis_name="c", subcore_axis_name="s")

    @functools.partial(
        pl.kernel,
        out_type=jax.ShapeDtypeStruct((N_SC, n_acc, d), jnp.float32),
        mesh=mesh,
        compiler_params=pltpu.CompilerParams(needs_layout_passes=False),
        scratch_types=[
            pltpu.VMEM((n_pad,), jnp.float32),
            pltpu.VMEM((den_rows, 128), jnp.float32),
            pltpu.VMEM((2 * SA,), jnp.int32),
            pltpu.VMEM((2 * SA,), jnp.int32),
            pltpu.VMEM((SB,), jnp.float32),
            pltpu.VMEM((4, 32, d), jnp.float32),
            pltpu.VMEM_SHARED((n_acc, d), jnp.float32),
            pltpu.VMEM_SHARED((den_rows, 128), jnp.float32),
            pltpu.SemaphoreType.DMA,
            pltpu.SemaphoreType.DMA,
            pltpu.SemaphoreType.DMA,
            pltpu.SemaphoreType.DMA,
            pltpu.SemaphoreType.DMA,
            pltpu.SemaphoreType.DMA,
            pltpu.SemaphoreType.DMA,
            pltpu.SemaphoreType.DMA,
            pltpu.SemaphoreType.DMA,
        ],
    )
    def sc_kernel(h_hbm, a_hbm, src_hbm, dst_hbm, out_hbm,
                  a_loc, den_loc, src_loc, dst_loc, w_loc, rows3,
                  out_acc, den_sh, sem_st, sem_g0, sem_g1, sem_g2, sem_g3,
                  sem_s0, sem_s1, sem_s2, sem_s3):
        sem_g = (sem_g0, sem_g1, sem_g2, sem_g3)
        sem_s = (sem_s0, sem_s1, sem_s2, sem_s3)
        c = lax.axis_index("c")
        s = lax.axis_index("s")
        z16 = jnp.zeros((LANES,), jnp.float32)

        pltpu.sync_copy(a_hbm, a_loc)

        def zden(i, carry):
            for j in range(128 // LANES):
                den_loc[i, pl.ds(j * LANES, LANES)] = z16
            return carry
        lax.fori_loop(0, den_rows, zden, 0)

        def zrow(i, carry):
            for bb in range(4):
                for j in range(d // LANES):
                    rows3[bb, i, pl.ds(j * LANES, LANES)] = z16
            return carry
        lax.fori_loop(0, 32, zrow, 0)

        @pl.when(s == 0)
        def _():
            pltpu.sync_copy(den_loc, den_sh)

        for k in range(zrows // 32):
            pltpu.sync_copy(rows3.at[0], out_acc.at[pl.ds(s * zrows + k * 32, 32)])
        rem = zrows % 32
        if rem:
            pltpu.sync_copy(rows3.at[0, pl.ds(0, rem)],
                            out_acc.at[pl.ds(s * zrows + zrows - rem, rem)])

        def edge_vals(i):
            si = src_loc[pl.ds(i * LANES, LANES)]
            di = dst_loc[pl.ds(i * LANES, LANES)]
            av = plsc.load_gather(a_loc, [si]) + plsc.load_gather(a_loc, [di])
            al = jnp.where(av >= 0.0, av, 0.2 * av)
            e = jnp.exp(al)
            r = lax.shift_right_logical(si, 7)
            lane = jnp.bitwise_and(si, 127)
            return e, r, lane

        sc_a = jax.named_scope("ph_passA"); sc_a.__enter__()
        n_blk_a = ch_a // SA

        def stage_a(t):
            half = lax.rem(t, 2) * SA
            base = s * ch_a + t * SA
            pltpu.async_copy(src_hbm.at[pl.ds(base, SA)],
                             src_loc.at[pl.ds(half, SA)], sem_st)
            pltpu.async_copy(dst_hbm.at[pl.ds(base, SA)],
                             dst_loc.at[pl.ds(half, SA)], sem_st)

        def wait_stage_a():
            pltpu.make_async_copy(src_hbm.at[pl.ds(0, SA)],
                                  src_loc.at[pl.ds(0, SA)], sem_st).wait()
            pltpu.make_async_copy(dst_hbm.at[pl.ds(0, SA)],
                                  dst_loc.at[pl.ds(0, SA)], sem_st).wait()

        stage_a(0)

        def pa_blk(t, carry):
            wait_stage_a()

            @pl.when(t < n_blk_a - 1)
            def _():
                stage_a(t + 1)
            hoff_a = lax.rem(t, 2) * SA

            def pa(i, carry2):
                e, r, lane = edge_vals(hoff_a // LANES + i)
                plsc.addupdate_scatter(den_loc, [r, lane], e)
                return carry2
            lax.fori_loop(0, SA // LANES, pa, 0, unroll=2)
            return carry
        lax.fori_loop(0, n_blk_a, pa_blk, 0)
        sc_a.__exit__(None, None, None)

        sc_r = jax.named_scope("ph_denred"); sc_r.__enter__()
        plsc.subcore_barrier()
        for k in range(den_rows // LANES):
            ivec = lax.iota(jnp.int32, LANES) + k * LANES
            pltpu.sync_copy(den_loc.at[pl.ds(k * LANES, LANES)],
                            den_sh.at[ivec], add=True)
        plsc.subcore_barrier()
        pltpu.sync_copy(den_sh, den_loc)
        sc_r.__exit__(None, None, None)
        sc_b = jax.named_scope("ph_passB"); sc_b.__enter__()

        wid = s * N_SC + c
        n_it = ch_b // SB
        b_base = wid * ch_b

        def stage(it, sync=False):
            slot = lax.rem(it, 4) * SB
            base = b_base + it * SB
            cp = pltpu.sync_copy if sync else (
                lambda a, b: pltpu.async_copy(a, b, sem_st))
            cp(src_hbm.at[pl.ds(base, SB)], src_loc.at[pl.ds(slot, SB)])
            cp(dst_hbm.at[pl.ds(base, SB)], dst_loc.at[pl.ds(slot, SB)])

        def wait_stage():
            pltpu.make_async_copy(src_hbm.at[pl.ds(0, SB)],
                                  src_loc.at[pl.ds(0, SB)], sem_st).wait()
            pltpu.make_async_copy(dst_hbm.at[pl.ds(0, SB)],
                                  dst_loc.at[pl.ds(0, SB)], sem_st).wait()

        def fire_gather(off, buf):
            pltpu.async_copy(h_hbm.at[src_loc.at[pl.ds(off, 32)]],
                             rows3.at[buf], sem_g[buf])

        def drain_gather(buf):
            pltpu.make_async_copy(h_hbm.at[pl.ds(0, 32)],
                                  rows3.at[buf], sem_g[buf]).wait()

        def drain_scat(buf):
            pltpu.make_async_copy(h_hbm.at[pl.ds(0, 32)],
                                  rows3.at[buf], sem_s[buf]).wait()

        stage(0, sync=True)
        stage(1)
        fire_gather(0, 0)
        fire_gather(32, 1)

        def pb_it(it, carry):
            hoff = lax.rem(it, 4) * SB
            hoff_n = lax.rem(it + 1, 4) * SB

            def wv_loop(i, carry2):
                e, r, lane = edge_vals(hoff // LANES + i)
                den = plsc.load_gather(den_loc, [r, lane])
                w_loc[pl.ds(i * LANES, LANES)] = e / (den + 1e-16)
                return carry2
            lax.fori_loop(0, SB // LANES, wv_loop, 0)

            for k in range(4):
                t = it * 4 + k
                bnx2 = (k + 2) % 4

                @pl.when(t >= 2)
                def _():
                    drain_scat(bnx2)

                if k < 2:
                    fire_gather(hoff + (k + 2) * 32, bnx2)
                else:
                    @pl.when(it < n_it - 1)
                    def _():
                        fire_gather(hoff_n + (k - 2) * 32, bnx2)

                if k == 0:
                    @pl.when(it < n_it - 2)
                    def _():
                        stage(it + 2)
                if k == 1:
                    @pl.when(it < n_it - 1)
                    def _():
                        wait_stage()

                drain_gather(k)

                for g in range(2):
                    wv16 = w_loc[pl.ds(k * 32 + g * LANES, LANES)]
                    for kk in range(LANES):
                        wv = wv16[kk]
                        for j in range(d // LANES):
                            sl = pl.ds(j * LANES, LANES)
                            rows3[k, g * LANES + kk, sl] = (
                                rows3[k, g * LANES + kk, sl] * wv)

                for k16 in range(2):
                    dvec = dst_loc[pl.ds(hoff + k * 32 + k16 * LANES, LANES)]
                    pltpu.async_copy(rows3.at[k, pl.ds(k16 * LANES, LANES)],
                                     out_acc.at[dvec], sem_s[k], add=True)
            return carry
        lax.fori_loop(0, n_it, pb_it, 0)

        drain_scat((4 * n_it - 2) % 4)
        drain_scat((4 * n_it - 1) % 4)
        sc_b.__exit__(None, None, None)

        plsc.subcore_barrier()
        pltpu.sync_copy(out_acc.at[pl.ds(s * zrows, zrows)],
                        out_hbm.at[c, pl.ds(s * zrows, zrows)])

    return sc_kernel


def kernel(x, edge_index, W, b, att):
    n, d = x.shape
    e0 = edge_index.shape[1]
    e1 = e0 + n

    n_pad = 10240
    n_acc = 10112
    step = 331776
    e_pad = ((e1 + step - 1) // step) * step

    loops = jnp.arange(n, dtype=edge_index.dtype)
    ei = jnp.concatenate([edge_index, jnp.stack([loops, loops])], axis=1)
    pad = n + jnp.arange(e_pad - e1, dtype=jnp.int32) % (n_acc - n)
    src = jnp.concatenate([ei[0], pad])
    dst = jnp.concatenate([ei[1], pad])

    x_p = jnp.concatenate([x, jnp.zeros((n_pad - n, d), x.dtype)])
    h, a2 = _linear_and_scores(x_p, W, b, att, n_pad, d)
    a = a2.reshape(n_pad)

    partials = _make_sc_kernel(n_pad, n_acc, d, e_pad)(h, a, src, dst)
    out = _combine(partials, n_acc, d)
    return (out[:n], edge_index)
